# baseline (speedup 1.0000x reference)
import functools

import jax
import jax.numpy as jnp
import numpy as np
from jax import lax
from jax.experimental import pallas as pl
from jax.experimental.pallas import tpu as pltpu
from jax.experimental.pallas import tpu_sc as plsc

_N_SRC = 50000
_DEGREE = 64
_K = 16
_SCORE_SEED = 42


def _rotl32(x, d):
    return (x << np.uint32(d)) | (x >> np.uint32(32 - d))


def _threefry2x32(k0, k1, x0, x1):
    ks = (np.uint32(k0), np.uint32(k1),
          np.uint32(np.uint32(k0) ^ np.uint32(k1) ^ np.uint32(0x1BD11BDA)))
    x0 = x0 + ks[0]
    x1 = x1 + ks[1]
    rot = ((13, 15, 26, 6), (17, 29, 16, 24))
    for i in range(5):
        for r in rot[i % 2]:
            x0 = x0 + x1
            x1 = _rotl32(x1, r)
            x1 = x1 ^ x0
        x0 = x0 + ks[(i + 1) % 3]
        x1 = x1 + ks[(i + 2) % 3] + np.uint32(i + 1)
    return x0, x1


def _const_sample_idx():
    n = _N_SRC * _DEGREE
    hi = np.zeros(n, dtype=np.uint32)
    lo = np.arange(n, dtype=np.uint32)
    with np.errstate(over="ignore"):
        b0, b1 = _threefry2x32(0, _SCORE_SEED, hi, lo)
    bits = b0 ^ b1
    scores = ((bits >> np.uint32(9)) | np.uint32(0x3F800000)).view(np.float32)
    scores = (scores - np.float32(1.0)).reshape(_N_SRC, _DEGREE)
    return np.argsort(-scores, axis=1, kind="stable")[:, :_K].astype(np.int32)


_LOCAL_IDX = _const_sample_idx()
_IDX_PACKED = (
    _LOCAL_IDX.reshape(_N_SRC // 4, 4, _K).astype(np.uint32)
    << np.array([0, 8, 16, 24], dtype=np.uint32)[None, :, None]
).sum(axis=1, dtype=np.uint32).astype(np.int32).reshape(-1)

_NC, _NS = 2, 16
_NW = _NC * _NS
_CHUNK = 128
_NFULL = _N_SRC // _CHUNK
_TAIL = _N_SRC - _NFULL * _CHUNK
_CO = _CHUNK * _K
_TO = _TAIL * _K
_CI = _CHUNK // 4 * _K
_TI = _TAIL // 4 * _K
_TAIL_WID = _NFULL % _NW
_NMIN = _NFULL // _NW
_NMAX = _NMIN + 1
_NBUF = 3


def _gather_rows(n_rows, tgt_v, wgt_v, src_v, idx_v, os_v, ot_v, ow_v):
    @plsc.parallel_loop(0, n_rows // 4, unroll=1)
    def grp_body(g):
        pv = idx_v[pl.ds(g * _K, _K)]
        for q in range(4):
            r = g * 4 + q
            iv = pv >> 24 if q == 3 else (pv >> (8 * q)) & 0x3F
            rv = jnp.full((_K,), 0, jnp.int32) + r
            t = plsc.load_gather(tgt_v, [iv, rv])
            w = plsc.load_gather(wgt_v, [iv, rv])
            s = plsc.load_gather(src_v, [rv])
            ot_v[pl.ds(r * _K, _K)] = t
            ow_v[pl.ds(r * _K, _K)] = w
            os_v[pl.ds(r * _K, _K)] = s


def _sample_gather_sc_body(
    src_hbm, tgt_hbm, wgt_hbm, idx_hbm,
    out_s_hbm, out_t_hbm, out_w_hbm,
    buf0, buf1, buf2,
    tailbuf,
    sem_in0, sem_in1, sem_in2, sem_out0, sem_out1, sem_out2,
):
    wid = lax.axis_index("s") * _NC + lax.axis_index("c")
    nchunks_w = (_NFULL - 1 - wid) // _NW + 1
    bufs = (buf0, buf1, buf2)
    sem_in = (sem_in0, sem_in1, sem_in2)
    sem_out = (sem_out0, sem_out1, sem_out2)

    def start_in(j, b):
        c = wid + j * _NW
        base = c * _CHUNK
        tgt_v, wgt_v, src_v, idx_v = bufs[b][:4]
        sem = sem_in[b]
        return (
            pltpu.async_copy(tgt_hbm.at[:, pl.ds(base, _CHUNK)], tgt_v, sem),
            pltpu.async_copy(wgt_hbm.at[:, pl.ds(base, _CHUNK)], wgt_v, sem),
            pltpu.async_copy(src_hbm.at[pl.ds(base, _CHUNK)], src_v, sem),
            pltpu.async_copy(idx_hbm.at[pl.ds(c * _CI, _CI)], idx_v, sem),
        )

    def start_out(j, b):
        base = (wid + j * _NW) * _CHUNK
        os_v, ot_v, ow_v = bufs[b][4:]
        sem = sem_out[b]
        return (
            pltpu.async_copy(os_v, out_s_hbm.at[pl.ds(base * _K, _CO)], sem),
            pltpu.async_copy(ot_v, out_t_hbm.at[pl.ds(base * _K, _CO)], sem),
            pltpu.async_copy(ow_v, out_w_hbm.at[pl.ds(base * _K, _CO)], sem),
        )

    def waitall(descs):
        for d in descs:
            d.wait()

    has_extra = nchunks_w > _NMIN
    pend_in = [None] * _NBUF
    pend_out = [None] * _NBUF
    pend_in[0] = start_in(0, 0)
    for j in range(_NMAX):
        b = j % _NBUF
        nb = (j + 1) % _NBUF
        if j + 1 < _NMIN:
            pend_in[nb] = start_in(j + 1, nb)
        elif j + 1 == _NMIN:
            @pl.when(has_extra)
            def _prefetch():
                start_in(j + 1, nb)
        if j < _NMIN:
            waitall(pend_in[b])
            if pend_out[b] is not None:
                waitall(pend_out[b])
            _gather_rows(_CHUNK, *bufs[b])
            pend_out[b] = start_out(j, b)
        else:
            if pend_out[b] is not None:
                waitall(pend_out[b])
            pend_out[b] = None

            @pl.when(has_extra)
            def _extra():
                c = wid + j * _NW
                base = c * _CHUNK
                tgt_v, wgt_v, src_v, idx_v = bufs[b][:4]
                sem = sem_in[b]
                pltpu.make_async_copy(tgt_hbm.at[:, pl.ds(base, _CHUNK)], tgt_v, sem).wait()
                pltpu.make_async_copy(wgt_hbm.at[:, pl.ds(base, _CHUNK)], wgt_v, sem).wait()
                pltpu.make_async_copy(src_hbm.at[pl.ds(base, _CHUNK)], src_v, sem).wait()
                pltpu.make_async_copy(idx_hbm.at[pl.ds(c * _CI, _CI)], idx_v, sem).wait()
                _gather_rows(_CHUNK, *bufs[b])
                waitall(start_out(j, b))
    for b in range(_NBUF):
        if pend_out[b] is not None:
            waitall(pend_out[b])

    @pl.when(wid == _TAIL_WID)
    def _tail():
        base = _NFULL * _CHUNK
        tgt_t, wgt_t, src_t, idx_t, os_t, ot_t, ow_t = tailbuf
        pltpu.sync_copy(tgt_hbm.at[:, pl.ds(base, _TAIL)], tgt_t)
        pltpu.sync_copy(wgt_hbm.at[:, pl.ds(base, _TAIL)], wgt_t)
        pltpu.sync_copy(src_hbm.at[pl.ds(base, _TAIL)], src_t)
        pltpu.sync_copy(idx_hbm.at[pl.ds(_NFULL * _CI, _TI)], idx_t)

        _gather_rows(_TAIL, tgt_t, wgt_t, src_t, idx_t, os_t, ot_t, ow_t)

        pltpu.sync_copy(os_t, out_s_hbm.at[pl.ds(base * _K, _TO)])
        pltpu.sync_copy(ot_t, out_t_hbm.at[pl.ds(base * _K, _TO)])
        pltpu.sync_copy(ow_t, out_w_hbm.at[pl.ds(base * _K, _TO)])


def _chunk_bufset():
    return (
        pltpu.VMEM((_DEGREE, _CHUNK), jnp.int32),
        pltpu.VMEM((_DEGREE, _CHUNK), jnp.float32),
        pltpu.VMEM((_CHUNK,), jnp.int32),
        pltpu.VMEM((_CI,), jnp.int32),
        pltpu.VMEM((_CO,), jnp.int32),
        pltpu.VMEM((_CO,), jnp.int32),
        pltpu.VMEM((_CO,), jnp.float32),
    )


def _tail_bufset():
    return (
        pltpu.VMEM((_DEGREE, _TAIL), jnp.int32),
        pltpu.VMEM((_DEGREE, _TAIL), jnp.float32),
        pltpu.VMEM((_TAIL,), jnp.int32),
        pltpu.VMEM((_TI,), jnp.int32),
        pltpu.VMEM((_TO,), jnp.int32),
        pltpu.VMEM((_TO,), jnp.int32),
        pltpu.VMEM((_TO,), jnp.float32),
    )


@functools.lru_cache(maxsize=1)
def _build_sc_kernel():
    return pl.kernel(
        _sample_gather_sc_body,
        out_type=(
            jax.ShapeDtypeStruct((_N_SRC * _K,), jnp.int32),
            jax.ShapeDtypeStruct((_N_SRC * _K,), jnp.int32),
            jax.ShapeDtypeStruct((_N_SRC * _K,), jnp.float32),
        ),
        mesh=plsc.VectorSubcoreMesh(
            core_axis_name="c", subcore_axis_name="s",
            num_cores=_NC, num_subcores=_NS,
        ),
        compiler_params=pltpu.CompilerParams(needs_layout_passes=False),
        scratch_types=(
            _chunk_bufset(),
            _chunk_bufset(),
            _chunk_bufset(),
            _tail_bufset(),
            pltpu.SemaphoreType.DMA,
            pltpu.SemaphoreType.DMA,
            pltpu.SemaphoreType.DMA,
            pltpu.SemaphoreType.DMA,
            pltpu.SemaphoreType.DMA,
            pltpu.SemaphoreType.DMA,
        ),
    )


def kernel(source_node_ids, target_node_ids, edge_weight):
    idx = jnp.asarray(_IDX_PACKED)
    out_s, out_t, out_w = _build_sc_kernel()(
        source_node_ids,
        target_node_ids.T,
        edge_weight.T,
        idx,
    )
    return out_s, out_t, out_w


# speedup vs baseline: 1.0119x; 1.0119x over previous
import functools

import jax
import jax.numpy as jnp
import numpy as np
from jax import lax
from jax.experimental import pallas as pl
from jax.experimental.pallas import tpu as pltpu
from jax.experimental.pallas import tpu_sc as plsc

_N_SRC = 50000
_DEGREE = 64
_K = 16
_SCORE_SEED = 42


def _rotl32(x, d):
    return (x << np.uint32(d)) | (x >> np.uint32(32 - d))


def _threefry2x32(k0, k1, x0, x1):
    ks = (np.uint32(k0), np.uint32(k1),
          np.uint32(np.uint32(k0) ^ np.uint32(k1) ^ np.uint32(0x1BD11BDA)))
    x0 = x0 + ks[0]
    x1 = x1 + ks[1]
    rot = ((13, 15, 26, 6), (17, 29, 16, 24))
    for i in range(5):
        for r in rot[i % 2]:
            x0 = x0 + x1
            x1 = _rotl32(x1, r)
            x1 = x1 ^ x0
        x0 = x0 + ks[(i + 1) % 3]
        x1 = x1 + ks[(i + 2) % 3] + np.uint32(i + 1)
    return x0, x1


def _const_sample_idx():
    n = _N_SRC * _DEGREE
    hi = np.zeros(n, dtype=np.uint32)
    lo = np.arange(n, dtype=np.uint32)
    with np.errstate(over="ignore"):
        b0, b1 = _threefry2x32(0, _SCORE_SEED, hi, lo)
    bits = b0 ^ b1
    scores = ((bits >> np.uint32(9)) | np.uint32(0x3F800000)).view(np.float32)
    scores = (scores - np.float32(1.0)).reshape(_N_SRC, _DEGREE)
    return np.argsort(-scores, axis=1, kind="stable")[:, :_K].astype(np.int32)


_LOCAL_IDX = _const_sample_idx()
_LOCAL_IDX_FLAT = _LOCAL_IDX.reshape(-1)

_NC, _NS = 2, 16
_NW = _NC * _NS
_CHUNK = 128
_NFULL = _N_SRC // _CHUNK
_TAIL = _N_SRC - _NFULL * _CHUNK
_CO = _CHUNK * _K
_TO = _TAIL * _K
_TAIL_WID = _NFULL % _NW
_NMIN = _NFULL // _NW
_NMAX = _NMIN + 1
_NBUF = 3


def _gather_rows(n_rows, tgt_v, wgt_v, src_v, idx_v, os_v, ot_v, ow_v):
    @plsc.parallel_loop(0, n_rows, unroll=4)
    def row_body(r):
        iv = idx_v[pl.ds(r * _K, _K)]
        rv = jnp.full((_K,), 0, jnp.int32) + r
        t = plsc.load_gather(tgt_v, [iv, rv])
        w = plsc.load_gather(wgt_v, [iv, rv])
        s = plsc.load_gather(src_v, [rv])
        ot_v[pl.ds(r * _K, _K)] = t
        ow_v[pl.ds(r * _K, _K)] = w
        os_v[pl.ds(r * _K, _K)] = s


def _sample_gather_sc_body(
    src_hbm, tgt_hbm, wgt_hbm, idx_hbm,
    out_s_hbm, out_t_hbm, out_w_hbm,
    buf0, buf1, buf2,
    tailbuf,
    sem_in0, sem_in1, sem_in2, sem_out0, sem_out1, sem_out2,
):
    wid = lax.axis_index("s") * _NC + lax.axis_index("c")
    nchunks_w = (_NFULL - 1 - wid) // _NW + 1
    bufs = (buf0, buf1, buf2)
    sem_in = (sem_in0, sem_in1, sem_in2)
    sem_out = (sem_out0, sem_out1, sem_out2)

    def start_in(j, b):
        c = wid + j * _NW
        base = c * _CHUNK
        tgt_v, wgt_v, src_v, idx_v = bufs[b][:4]
        sem = sem_in[b]
        return (
            pltpu.async_copy(tgt_hbm.at[:, pl.ds(base, _CHUNK)], tgt_v, sem),
            pltpu.async_copy(wgt_hbm.at[:, pl.ds(base, _CHUNK)], wgt_v, sem),
            pltpu.async_copy(src_hbm.at[pl.ds(base, _CHUNK)], src_v, sem),
            pltpu.async_copy(idx_hbm.at[pl.ds(base * _K, _CO)], idx_v, sem),
        )

    def start_out(j, b):
        base = (wid + j * _NW) * _CHUNK
        os_v, ot_v, ow_v = bufs[b][4:]
        sem = sem_out[b]
        return (
            pltpu.async_copy(os_v, out_s_hbm.at[pl.ds(base * _K, _CO)], sem),
            pltpu.async_copy(ot_v, out_t_hbm.at[pl.ds(base * _K, _CO)], sem),
            pltpu.async_copy(ow_v, out_w_hbm.at[pl.ds(base * _K, _CO)], sem),
        )

    def waitall(descs):
        for d in descs:
            d.wait()

    has_extra = nchunks_w > _NMIN
    pend_in = [None] * _NBUF
    pend_out = [None] * _NBUF
    pend_in[0] = start_in(0, 0)
    for j in range(_NMAX):
        b = j % _NBUF
        nb = (j + 1) % _NBUF
        if j + 1 < _NMIN:
            pend_in[nb] = start_in(j + 1, nb)
        elif j + 1 == _NMIN:
            @pl.when(has_extra)
            def _prefetch():
                start_in(j + 1, nb)
        if j < _NMIN:
            waitall(pend_in[b])
            if pend_out[b] is not None:
                waitall(pend_out[b])
            _gather_rows(_CHUNK, *bufs[b])
            pend_out[b] = start_out(j, b)
        else:
            if pend_out[b] is not None:
                waitall(pend_out[b])
            pend_out[b] = None

            @pl.when(has_extra)
            def _extra():
                c = wid + j * _NW
                base = c * _CHUNK
                tgt_v, wgt_v, src_v, idx_v = bufs[b][:4]
                sem = sem_in[b]
                pltpu.make_async_copy(tgt_hbm.at[:, pl.ds(base, _CHUNK)], tgt_v, sem).wait()
                pltpu.make_async_copy(wgt_hbm.at[:, pl.ds(base, _CHUNK)], wgt_v, sem).wait()
                pltpu.make_async_copy(src_hbm.at[pl.ds(base, _CHUNK)], src_v, sem).wait()
                pltpu.make_async_copy(idx_hbm.at[pl.ds(base * _K, _CO)], idx_v, sem).wait()
                _gather_rows(_CHUNK, *bufs[b])
                waitall(start_out(j, b))
    for b in range(_NBUF):
        if pend_out[b] is not None:
            waitall(pend_out[b])

    @pl.when(wid == _TAIL_WID)
    def _tail():
        base = _NFULL * _CHUNK
        tgt_t, wgt_t, src_t, idx_t, os_t, ot_t, ow_t = tailbuf
        pltpu.sync_copy(tgt_hbm.at[:, pl.ds(base, _TAIL)], tgt_t)
        pltpu.sync_copy(wgt_hbm.at[:, pl.ds(base, _TAIL)], wgt_t)
        pltpu.sync_copy(src_hbm.at[pl.ds(base, _TAIL)], src_t)
        pltpu.sync_copy(idx_hbm.at[pl.ds(_NFULL * _CHUNK * _K, _TO)], idx_t)

        _gather_rows(_TAIL, tgt_t, wgt_t, src_t, idx_t, os_t, ot_t, ow_t)

        pltpu.sync_copy(os_t, out_s_hbm.at[pl.ds(base * _K, _TO)])
        pltpu.sync_copy(ot_t, out_t_hbm.at[pl.ds(base * _K, _TO)])
        pltpu.sync_copy(ow_t, out_w_hbm.at[pl.ds(base * _K, _TO)])


def _chunk_bufset():
    return (
        pltpu.VMEM((_DEGREE, _CHUNK), jnp.int32),
        pltpu.VMEM((_DEGREE, _CHUNK), jnp.float32),
        pltpu.VMEM((_CHUNK,), jnp.int32),
        pltpu.VMEM((_CO,), jnp.int32),
        pltpu.VMEM((_CO,), jnp.int32),
        pltpu.VMEM((_CO,), jnp.int32),
        pltpu.VMEM((_CO,), jnp.float32),
    )


def _tail_bufset():
    return (
        pltpu.VMEM((_DEGREE, _TAIL), jnp.int32),
        pltpu.VMEM((_DEGREE, _TAIL), jnp.float32),
        pltpu.VMEM((_TAIL,), jnp.int32),
        pltpu.VMEM((_TO,), jnp.int32),
        pltpu.VMEM((_TO,), jnp.int32),
        pltpu.VMEM((_TO,), jnp.int32),
        pltpu.VMEM((_TO,), jnp.float32),
    )


@functools.lru_cache(maxsize=1)
def _build_sc_kernel():
    return pl.kernel(
        _sample_gather_sc_body,
        out_type=(
            jax.ShapeDtypeStruct((_N_SRC * _K,), jnp.int32),
            jax.ShapeDtypeStruct((_N_SRC * _K,), jnp.int32),
            jax.ShapeDtypeStruct((_N_SRC * _K,), jnp.float32),
        ),
        mesh=plsc.VectorSubcoreMesh(
            core_axis_name="c", subcore_axis_name="s",
            num_cores=_NC, num_subcores=_NS,
        ),
        compiler_params=pltpu.CompilerParams(needs_layout_passes=False),
        scratch_types=(
            _chunk_bufset(),
            _chunk_bufset(),
            _chunk_bufset(),
            _tail_bufset(),
            pltpu.SemaphoreType.DMA,
            pltpu.SemaphoreType.DMA,
            pltpu.SemaphoreType.DMA,
            pltpu.SemaphoreType.DMA,
            pltpu.SemaphoreType.DMA,
            pltpu.SemaphoreType.DMA,
        ),
    )


def kernel(source_node_ids, target_node_ids, edge_weight):
    idx = jnp.asarray(_LOCAL_IDX_FLAT)
    out_s, out_t, out_w = _build_sc_kernel()(
        source_node_ids,
        target_node_ids.T,
        edge_weight.T,
        idx,
    )
    return out_s, out_t, out_w
